# baseline (speedup 1.0000x reference)
import functools

import jax
import jax.numpy as jnp
from jax import lax
from jax.experimental import pallas as pl
from jax.experimental.pallas import tpu as pltpu
from jax.experimental.pallas import tpu_sc as plsc


def _sc_relayout(table_t, tail_lin):
    D, V = table_t.shape
    TL = 128
    NT_FULL = V // TL
    TAIL = V - NT_FULL * TL
    info = plsc.get_sparse_core_info()
    NC, NS = info.num_cores, info.num_subcores
    NW = NC * NS
    PER_W = (NT_FULL + NW - 1) // NW

    mesh = plsc.VectorSubcoreMesh(core_axis_name="c", subcore_axis_name="s")

    @functools.partial(
        pl.kernel,
        mesh=mesh,
        out_type=jax.ShapeDtypeStruct((V * D,), jnp.float32),
        scratch_types=[
            pltpu.VMEM((2, D, TL), jnp.float32),
            pltpu.VMEM((2, D * TL), jnp.float32),
            pltpu.SemaphoreType.DMA,
            pltpu.SemaphoreType.DMA,
            pltpu.SemaphoreType.DMA,
            pltpu.SemaphoreType.DMA,
        ],
        compiler_params=pltpu.CompilerParams(use_tc_tiling_on_sc=True,
                                             needs_layout_passes=False),
    )
    def k(tab_hbm, tail_hbm, out_hbm, in_v, out_v,
          sem_i0, sem_i1, sem_o0, sem_o1):
        wid = lax.axis_index("s") * NC + lax.axis_index("c")

        @pl.when(wid == 0)
        def _():
            pltpu.sync_copy(tail_hbm, out_v.at[0, pl.ds(0, TAIL * D)])
            pltpu.sync_copy(out_v.at[0, pl.ds(0, TAIL * D)],
                            out_hbm.at[pl.ds(NT_FULL * TL * D, TAIL * D)])
        isems = (sem_i0, sem_i1)
        osems = (sem_o0, sem_o1)
        lanes = lax.broadcasted_iota(jnp.int32, (16,), 0)

        def tile_of(i):
            return wid + NW * i

        def start_in(t, buf):
            pltpu.async_copy(tab_hbm.at[:, pl.ds(t * TL, TL)],
                             in_v.at[buf], isems[buf])

        def wait_in(t, buf):
            pltpu.make_async_copy(tab_hbm.at[:, pl.ds(t * TL, TL)],
                                  in_v.at[buf], isems[buf]).wait()

        def transpose_tile(buf):
            for a in range(D):
                for g in range(8):
                    dvec = lanes + (16 * (g % 2))
                    col = jnp.int32(4 * a + (g // 2))
                    vals = plsc.load_gather(
                        in_v.at[buf],
                        [dvec, lax.broadcast_in_dim(col, (16,), ())])
                    out_v[buf, pl.ds(128 * a + 16 * g, 16)] = vals

        def start_out(t, buf):
            pltpu.async_copy(out_v.at[buf],
                             out_hbm.at[pl.ds(t * TL * D, TL * D)],
                             osems[buf])

        def wait_out(t, buf):
            pltpu.make_async_copy(out_v.at[buf],
                                  out_hbm.at[pl.ds(t * TL * D, TL * D)],
                                  osems[buf]).wait()

        @pl.when(tile_of(0) < NT_FULL)
        def _():
            start_in(tile_of(0), 0)

        def do_tile(i, buf):
            t = tile_of(i)

            @pl.when((i >= 2) & (tile_of(i - 2) < NT_FULL))
            def _():
                wait_out(tile_of(i - 2), buf)

            @pl.when(t < NT_FULL)
            def _():
                tn = tile_of(i + 1)

                @pl.when(tn < NT_FULL)
                def _():
                    start_in(tn, 1 - buf)
                wait_in(t, buf)
                transpose_tile(buf)
                start_out(t, buf)

        n_j = (PER_W + 2) // 2

        def step(j, carry):
            do_tile(2 * j, 0)
            do_tile(2 * j + 1, 1)
            return carry

        lax.fori_loop(0, n_j, step, 0, unroll=False)
        for i in (2 * n_j - 2, 2 * n_j - 1):
            @pl.when(tile_of(i) < NT_FULL)
            def _():
                wait_out(tile_of(i), i % 2)

    return k(table_t, tail_lin)


def _sc_attend(item_input, hist_t, item_table):
    B = item_input.shape[0]
    L = hist_t.shape[0]
    D = item_table.shape[1]
    info = plsc.get_sparse_core_info()
    NC, NS = info.num_cores, info.num_subcores
    NW = NC * NS
    b_per_w = B // NW
    C0, C1 = 128, L - 128
    LP = 16 * ((L + 15) // 16)

    mesh = plsc.VectorSubcoreMesh(core_axis_name="c", subcore_axis_name="s")

    @functools.partial(
        pl.kernel,
        mesh=mesh,
        out_type=(
            jax.ShapeDtypeStruct((B, D), jnp.float32),
            jax.ShapeDtypeStruct((B, D), jnp.float32),
        ),
        scratch_types=[
            pltpu.VMEM((b_per_w,), jnp.int32),
            pltpu.VMEM((b_per_w, D), jnp.float32),
            pltpu.VMEM((L, b_per_w), jnp.int32),
            pltpu.VMEM((b_per_w, LP), jnp.int32),
            pltpu.VMEM((2, L, D), jnp.float32),
            pltpu.VMEM((b_per_w, D), jnp.float32),
            pltpu.SemaphoreType.DMA,
            pltpu.SemaphoreType.DMA,
            pltpu.SemaphoreType.DMA,
        ],
        compiler_params=pltpu.CompilerParams(use_tc_tiling_on_sc=False,
                                             needs_layout_passes=False),
    )
    def k(iidx_hbm, hidx_hbm, itab_hbm, iout, wout,
          iidx_v, irows_v, hcol_v, hidx_v, hrows_v, wout_v,
          sem_i, sem_h0, sem_h1):
        wid = lax.axis_index("s") * NC + lax.axis_index("c")
        base = wid * b_per_w

        pltpu.sync_copy(iidx_hbm.at[pl.ds(base, b_per_w)], iidx_v)
        cp_i = pltpu.async_copy(itab_hbm.at[iidx_v], irows_v, sem_i)
        pltpu.sync_copy(hidx_hbm.at[:, pl.ds(base, b_per_w)], hcol_v)
        lanes = lax.broadcasted_iota(jnp.int32, (16,), 0)

        def transpose_b(b, carry):
            bvec = lax.broadcast_in_dim(b, (16,), ()).astype(jnp.int32)
            for lg in range(LP // 16):
                lvec = lanes + (16 * lg)
                mask = lvec < L
                vals = plsc.load_gather(hcol_v, [lvec, bvec], mask=mask)
                hidx_v[b, pl.ds(16 * lg, 16)] = vals
            return carry

        lax.fori_loop(0, b_per_w, transpose_b, 0, unroll=False)

        sems = (sem_h0, sem_h1)

        def start(b, buf):
            pltpu.async_copy(itab_hbm.at[hidx_v.at[b, pl.ds(0, C0)]],
                             hrows_v.at[buf, pl.ds(0, C0)], sems[buf])
            pltpu.async_copy(itab_hbm.at[hidx_v.at[b, pl.ds(C0, C1)]],
                             hrows_v.at[buf, pl.ds(C0, C1)], sems[buf])

        def drain(b, buf):
            pltpu.make_async_copy(itab_hbm.at[hidx_v.at[b, pl.ds(0, C0)]],
                                  hrows_v.at[buf, pl.ds(0, C0)],
                                  sems[buf]).wait()
            pltpu.make_async_copy(itab_hbm.at[hidx_v.at[b, pl.ds(C0, C1)]],
                                  hrows_v.at[buf, pl.ds(C0, C1)],
                                  sems[buf]).wait()

        start(0, 0)
        cp_i.wait()
        pltpu.sync_copy(irows_v, iout.at[pl.ds(base, b_per_w)])

        def compute_row(b, buf):
            q0 = irows_v[b, pl.ds(0, 16)]
            q1 = irows_v[b, pl.ds(16, 16)]
            zero = jnp.zeros((16,), jnp.float32)

            def group(g, carry):
                a0x, a1x, zx, a0y, a1y, zy = carry
                accs = [[a0x, a1x, zx], [a0y, a1y, zy]]
                for kk in range(8):
                    l = g * 8 + kk
                    h0 = hrows_v[buf, l, pl.ds(0, 16)]
                    h1 = hrows_v[buf, l, pl.ds(16, 16)]
                    s = jnp.sum(h0 * q0 + h1 * q1)
                    e = jnp.exp(lax.broadcast_in_dim(s, (16,), ()))
                    a = accs[kk % 2]
                    a[0] = a[0] + e * h0
                    a[1] = a[1] + e * h1
                    a[2] = a[2] + e
                return (accs[0][0], accs[0][1], accs[0][2],
                        accs[1][0], accs[1][1], accs[1][2])

            a0x, a1x, zx, a0y, a1y, zy = lax.fori_loop(
                0, L // 8, group, (zero, zero, zero, zero, zero, zero),
                unroll=False)
            winv = 1.0 / (zx + zy)
            wout_v[b, pl.ds(0, 16)] = (a0x + a0y) * winv
            wout_v[b, pl.ds(16, 16)] = (a1x + a1y) * winv

        def step(t, carry):
            b = 2 * t
            start(b + 1, 1)
            drain(b, 0)
            compute_row(b, 0)
            start(b + 2, 0)
            drain(b + 1, 1)
            compute_row(b + 1, 1)
            return carry

        lax.fori_loop(0, (b_per_w - 2) // 2, step, 0, unroll=False)
        b = b_per_w - 2
        start(b + 1, 1)
        drain(b, 0)
        compute_row(b, 0)
        drain(b + 1, 1)
        compute_row(b + 1, 1)

        pltpu.sync_copy(wout_v, wout.at[pl.ds(base, b_per_w)])

    return k(item_input, hist_t, item_table)


def _mlp_body(u_ref, i_ref, w_ref, w1_ref, b1_ref, w2_ref, b2_ref,
              w3_ref, b3_ref, wo_ref, bo_ref, out_ref):
    w1 = w1_ref[...]
    x = (jnp.dot(u_ref[...], w1[0:32], preferred_element_type=jnp.float32)
         + jnp.dot(i_ref[...], w1[32:64], preferred_element_type=jnp.float32)
         + jnp.dot(w_ref[...], w1[64:96], preferred_element_type=jnp.float32)
         + b1_ref[...])
    x = jnp.maximum(x, 0.0)
    x = jnp.maximum(jnp.dot(x, w2_ref[...],
                            preferred_element_type=jnp.float32) + b2_ref[...], 0.0)
    x = jnp.maximum(jnp.dot(x, w3_ref[...],
                            preferred_element_type=jnp.float32) + b3_ref[...], 0.0)
    y = jnp.dot(x, wo_ref[...], preferred_element_type=jnp.float32) + bo_ref[...]
    out_ref[...] = jax.nn.sigmoid(y)


def _mlp(user_emb, item_emb, wh, W1, b1, W2, b2, W3, b3, Wo, bo):
    B, D = user_emb.shape
    full = lambda *s: pl.BlockSpec(s, lambda: (0,) * len(s))
    return pl.pallas_call(
        _mlp_body,
        in_specs=[
            full(B, D), full(B, D), full(B, D),
            full(*W1.shape), full(1, b1.shape[0]),
            full(*W2.shape), full(1, b2.shape[0]),
            full(*W3.shape), full(1, b3.shape[0]),
            full(*Wo.shape), full(1, bo.shape[0]),
        ],
        out_specs=full(B, 1),
        out_shape=jax.ShapeDtypeStruct((B, 1), jnp.float32),
    )(user_emb, item_emb, wh, W1, b1.reshape(1, -1), W2,
      b2.reshape(1, -1), W3, b3.reshape(1, -1), Wo, bo.reshape(1, -1))


@jax.jit
def kernel(user_input, item_input, history_items, user_table, item_table,
           W1, b1, W2, b2, W3, b3, Wo, bo):
    hist_t = jnp.transpose(history_items).astype(jnp.int32)
    user_emb = jnp.take(user_table, user_input, axis=0)
    n_tail = item_table.shape[0] % 128
    tail_lin = item_table[item_table.shape[0] - n_tail:].reshape(-1)
    itab_flat = _sc_relayout(jnp.transpose(item_table), tail_lin)
    itab_lin = itab_flat.reshape(item_table.shape)
    item_emb, wh = _sc_attend(item_input.astype(jnp.int32), hist_t,
                              itab_lin)
    return _mlp(user_emb, item_emb, wh, W1, b1, W2, b2, W3, b3, Wo, bo)


# speedup vs baseline: 1.1947x; 1.1947x over previous
import functools

import jax
import jax.numpy as jnp
from jax import lax
from jax.experimental import pallas as pl
from jax.experimental.pallas import tpu as pltpu
from jax.experimental.pallas import tpu_sc as plsc


def _sc_relayout(table_t, tail_lin):
    D, V = table_t.shape
    TL = 128
    NT_FULL = V // TL
    TAIL = V - NT_FULL * TL
    info = plsc.get_sparse_core_info()
    NC, NS = info.num_cores, info.num_subcores
    NW = NC * NS
    PER_W = (NT_FULL + NW - 1) // NW

    mesh = plsc.VectorSubcoreMesh(core_axis_name="c", subcore_axis_name="s")

    @functools.partial(
        pl.kernel,
        mesh=mesh,
        out_type=jax.ShapeDtypeStruct((V * D,), jnp.float32),
        scratch_types=[
            pltpu.VMEM((2, D, TL), jnp.float32),
            pltpu.VMEM((D * 129 + 16,), jnp.float32),
            pltpu.VMEM((2, D * TL), jnp.float32),
            pltpu.SemaphoreType.DMA,
            pltpu.SemaphoreType.DMA,
            pltpu.SemaphoreType.DMA,
            pltpu.SemaphoreType.DMA,
        ],
        compiler_params=pltpu.CompilerParams(use_tc_tiling_on_sc=True,
                                             needs_layout_passes=False),
    )
    def k(tab_hbm, tail_hbm, out_hbm, in_v, pad_v, out_v,
          sem_i0, sem_i1, sem_o0, sem_o1):
        wid = lax.axis_index("s") * NC + lax.axis_index("c")

        @pl.when(wid == 0)
        def _():
            pltpu.sync_copy(tail_hbm, out_v.at[0, pl.ds(0, TAIL * D)])
            pltpu.sync_copy(out_v.at[0, pl.ds(0, TAIL * D)],
                            out_hbm.at[pl.ds(NT_FULL * TL * D, TAIL * D)])
        isems = (sem_i0, sem_i1)
        osems = (sem_o0, sem_o1)
        lanes = lax.broadcasted_iota(jnp.int32, (16,), 0)

        def tile_of(i):
            return wid + NW * i

        def start_in(t, buf):
            pltpu.async_copy(tab_hbm.at[:, pl.ds(t * TL, TL)],
                             in_v.at[buf], isems[buf])

        def wait_in(t, buf):
            pltpu.make_async_copy(tab_hbm.at[:, pl.ds(t * TL, TL)],
                                  in_v.at[buf], isems[buf]).wait()

        def transpose_tile(buf):
            for d in range(D):
                for c in range(TL // 16):
                    v = in_v[buf, d, pl.ds(16 * c, 16)]
                    plsc.store_scatter(
                        pad_v, [lanes + (d * 129 + 16 * c)], v)
            for a in range(D):
                for g in range(8):
                    idx = (lanes + (16 * (g % 2))) * 129 + (4 * a + g // 2)
                    vals = plsc.load_gather(pad_v, [idx])
                    out_v[buf, pl.ds(128 * a + 16 * g, 16)] = vals

        def start_out(t, buf):
            pltpu.async_copy(out_v.at[buf],
                             out_hbm.at[pl.ds(t * TL * D, TL * D)],
                             osems[buf])

        def wait_out(t, buf):
            pltpu.make_async_copy(out_v.at[buf],
                                  out_hbm.at[pl.ds(t * TL * D, TL * D)],
                                  osems[buf]).wait()

        @pl.when(tile_of(0) < NT_FULL)
        def _():
            start_in(tile_of(0), 0)

        def do_tile(i, buf):
            t = tile_of(i)

            @pl.when((i >= 2) & (tile_of(i - 2) < NT_FULL))
            def _():
                wait_out(tile_of(i - 2), buf)

            @pl.when(t < NT_FULL)
            def _():
                tn = tile_of(i + 1)

                @pl.when(tn < NT_FULL)
                def _():
                    start_in(tn, 1 - buf)
                wait_in(t, buf)
                transpose_tile(buf)
                start_out(t, buf)

        n_j = (PER_W + 2) // 2

        def step(j, carry):
            do_tile(2 * j, 0)
            do_tile(2 * j + 1, 1)
            return carry

        lax.fori_loop(0, n_j, step, 0, unroll=False)
        for i in (2 * n_j - 2, 2 * n_j - 1):
            @pl.when(tile_of(i) < NT_FULL)
            def _():
                wait_out(tile_of(i), i % 2)

    return k(table_t, tail_lin)


def _sc_attend(item_input, hist_t, item_table):
    B = item_input.shape[0]
    L = hist_t.shape[0]
    D = item_table.shape[1]
    info = plsc.get_sparse_core_info()
    NC, NS = info.num_cores, info.num_subcores
    NW = NC * NS
    b_per_w = B // NW
    C0, C1 = 128, L - 128
    LP = 16 * ((L + 15) // 16)

    mesh = plsc.VectorSubcoreMesh(core_axis_name="c", subcore_axis_name="s")

    @functools.partial(
        pl.kernel,
        mesh=mesh,
        out_type=(
            jax.ShapeDtypeStruct((B, D), jnp.float32),
            jax.ShapeDtypeStruct((B, D), jnp.float32),
        ),
        scratch_types=[
            pltpu.VMEM((b_per_w,), jnp.int32),
            pltpu.VMEM((b_per_w, D), jnp.float32),
            pltpu.VMEM((L, b_per_w), jnp.int32),
            pltpu.VMEM((b_per_w, LP), jnp.int32),
            pltpu.VMEM((2, L, D), jnp.float32),
            pltpu.VMEM((b_per_w, D), jnp.float32),
            pltpu.SemaphoreType.DMA,
            pltpu.SemaphoreType.DMA,
            pltpu.SemaphoreType.DMA,
        ],
        compiler_params=pltpu.CompilerParams(use_tc_tiling_on_sc=False,
                                             needs_layout_passes=False),
    )
    def k(iidx_hbm, hidx_hbm, itab_hbm, iout, wout,
          iidx_v, irows_v, hcol_v, hidx_v, hrows_v, wout_v,
          sem_i, sem_h0, sem_h1):
        wid = lax.axis_index("s") * NC + lax.axis_index("c")
        base = wid * b_per_w

        pltpu.sync_copy(iidx_hbm.at[pl.ds(base, b_per_w)], iidx_v)
        cp_i = pltpu.async_copy(itab_hbm.at[iidx_v], irows_v, sem_i)
        pltpu.sync_copy(hidx_hbm.at[:, pl.ds(base, b_per_w)], hcol_v)
        lanes = lax.broadcasted_iota(jnp.int32, (16,), 0)

        def transpose_b(b, carry):
            bvec = lax.broadcast_in_dim(b, (16,), ()).astype(jnp.int32)
            for lg in range(LP // 16):
                lvec = lanes + (16 * lg)
                mask = lvec < L
                vals = plsc.load_gather(hcol_v, [lvec, bvec], mask=mask)
                hidx_v[b, pl.ds(16 * lg, 16)] = vals
            return carry

        lax.fori_loop(0, b_per_w, transpose_b, 0, unroll=False)

        sems = (sem_h0, sem_h1)

        def start(b, buf):
            pltpu.async_copy(itab_hbm.at[hidx_v.at[b, pl.ds(0, C0)]],
                             hrows_v.at[buf, pl.ds(0, C0)], sems[buf])
            pltpu.async_copy(itab_hbm.at[hidx_v.at[b, pl.ds(C0, C1)]],
                             hrows_v.at[buf, pl.ds(C0, C1)], sems[buf])

        def drain(b, buf):
            pltpu.make_async_copy(itab_hbm.at[hidx_v.at[b, pl.ds(0, C0)]],
                                  hrows_v.at[buf, pl.ds(0, C0)],
                                  sems[buf]).wait()
            pltpu.make_async_copy(itab_hbm.at[hidx_v.at[b, pl.ds(C0, C1)]],
                                  hrows_v.at[buf, pl.ds(C0, C1)],
                                  sems[buf]).wait()

        start(0, 0)
        cp_i.wait()
        pltpu.sync_copy(irows_v, iout.at[pl.ds(base, b_per_w)])

        def compute_row(b, buf):
            q0 = irows_v[b, pl.ds(0, 16)]
            q1 = irows_v[b, pl.ds(16, 16)]
            zero = jnp.zeros((16,), jnp.float32)

            def group(g, carry):
                a0x, a1x, zx, a0y, a1y, zy = carry
                accs = [[a0x, a1x, zx], [a0y, a1y, zy]]
                for kk in range(8):
                    l = g * 8 + kk
                    h0 = hrows_v[buf, l, pl.ds(0, 16)]
                    h1 = hrows_v[buf, l, pl.ds(16, 16)]
                    s = jnp.sum(h0 * q0 + h1 * q1)
                    e = jnp.exp(lax.broadcast_in_dim(s, (16,), ()))
                    a = accs[kk % 2]
                    a[0] = a[0] + e * h0
                    a[1] = a[1] + e * h1
                    a[2] = a[2] + e
                return (accs[0][0], accs[0][1], accs[0][2],
                        accs[1][0], accs[1][1], accs[1][2])

            a0x, a1x, zx, a0y, a1y, zy = lax.fori_loop(
                0, L // 8, group, (zero, zero, zero, zero, zero, zero),
                unroll=False)
            winv = 1.0 / (zx + zy)
            wout_v[b, pl.ds(0, 16)] = (a0x + a0y) * winv
            wout_v[b, pl.ds(16, 16)] = (a1x + a1y) * winv

        def step(t, carry):
            b = 2 * t
            start(b + 1, 1)
            drain(b, 0)
            compute_row(b, 0)
            start(b + 2, 0)
            drain(b + 1, 1)
            compute_row(b + 1, 1)
            return carry

        lax.fori_loop(0, (b_per_w - 2) // 2, step, 0, unroll=False)
        b = b_per_w - 2
        start(b + 1, 1)
        drain(b, 0)
        compute_row(b, 0)
        drain(b + 1, 1)
        compute_row(b + 1, 1)

        pltpu.sync_copy(wout_v, wout.at[pl.ds(base, b_per_w)])

    return k(item_input, hist_t, item_table)


def _mlp_body(u_ref, i_ref, w_ref, w1_ref, b1_ref, w2_ref, b2_ref,
              w3_ref, b3_ref, wo_ref, bo_ref, out_ref):
    w1 = w1_ref[...]
    x = (jnp.dot(u_ref[...], w1[0:32], preferred_element_type=jnp.float32)
         + jnp.dot(i_ref[...], w1[32:64], preferred_element_type=jnp.float32)
         + jnp.dot(w_ref[...], w1[64:96], preferred_element_type=jnp.float32)
         + b1_ref[...])
    x = jnp.maximum(x, 0.0)
    x = jnp.maximum(jnp.dot(x, w2_ref[...],
                            preferred_element_type=jnp.float32) + b2_ref[...], 0.0)
    x = jnp.maximum(jnp.dot(x, w3_ref[...],
                            preferred_element_type=jnp.float32) + b3_ref[...], 0.0)
    y = jnp.dot(x, wo_ref[...], preferred_element_type=jnp.float32) + bo_ref[...]
    out_ref[...] = jax.nn.sigmoid(y)


def _mlp(user_emb, item_emb, wh, W1, b1, W2, b2, W3, b3, Wo, bo):
    B, D = user_emb.shape
    full = lambda *s: pl.BlockSpec(s, lambda: (0,) * len(s))
    return pl.pallas_call(
        _mlp_body,
        in_specs=[
            full(B, D), full(B, D), full(B, D),
            full(*W1.shape), full(1, b1.shape[0]),
            full(*W2.shape), full(1, b2.shape[0]),
            full(*W3.shape), full(1, b3.shape[0]),
            full(*Wo.shape), full(1, bo.shape[0]),
        ],
        out_specs=full(B, 1),
        out_shape=jax.ShapeDtypeStruct((B, 1), jnp.float32),
    )(user_emb, item_emb, wh, W1, b1.reshape(1, -1), W2,
      b2.reshape(1, -1), W3, b3.reshape(1, -1), Wo, bo.reshape(1, -1))


@jax.jit
def kernel(user_input, item_input, history_items, user_table, item_table,
           W1, b1, W2, b2, W3, b3, Wo, bo):
    hist_t = jnp.transpose(history_items).astype(jnp.int32)
    user_emb = jnp.take(user_table, user_input, axis=0)
    n_tail = item_table.shape[0] % 128
    tail_lin = item_table[item_table.shape[0] - n_tail:].reshape(-1)
    itab_flat = _sc_relayout(jnp.transpose(item_table), tail_lin)
    itab_lin = itab_flat.reshape(item_table.shape)
    item_emb, wh = _sc_attend(item_input.astype(jnp.int32), hist_t,
                              itab_lin)
    return _mlp(user_emb, item_emb, wh, W1, b1, W2, b2, W3, b3, Wo, bo)


# speedup vs baseline: 1.7882x; 1.4968x over previous
import functools

import jax
import jax.numpy as jnp
from jax import lax
from jax.experimental import pallas as pl
from jax.experimental.pallas import tpu as pltpu
from jax.experimental.pallas import tpu_sc as plsc


def _sc_relayout(table_t, tail_lin):
    D, V = table_t.shape
    TL = 128
    NT_FULL = V // TL
    TAIL = V - NT_FULL * TL
    info = plsc.get_sparse_core_info()
    NC, NS = info.num_cores, info.num_subcores
    NW = NC * NS
    PER_W = (NT_FULL + NW - 1) // NW

    mesh = plsc.VectorSubcoreMesh(core_axis_name="c", subcore_axis_name="s")

    @functools.partial(
        pl.kernel,
        mesh=mesh,
        out_type=jax.ShapeDtypeStruct((V * D,), jnp.float32),
        scratch_types=[
            pltpu.VMEM((2, D, TL), jnp.float32),
            pltpu.VMEM((D * 129 + 16,), jnp.float32),
            pltpu.VMEM((2, D * TL), jnp.float32),
            pltpu.SemaphoreType.DMA,
            pltpu.SemaphoreType.DMA,
            pltpu.SemaphoreType.DMA,
            pltpu.SemaphoreType.DMA,
        ],
        compiler_params=pltpu.CompilerParams(use_tc_tiling_on_sc=True,
                                             needs_layout_passes=False),
    )
    def k(tab_hbm, tail_hbm, out_hbm, in_v, pad_v, out_v,
          sem_i0, sem_i1, sem_o0, sem_o1):
        wid = lax.axis_index("s") * NC + lax.axis_index("c")

        @pl.when(wid == 0)
        def _():
            pltpu.sync_copy(tail_hbm, out_v.at[0, pl.ds(0, TAIL * D)])
            pltpu.sync_copy(out_v.at[0, pl.ds(0, TAIL * D)],
                            out_hbm.at[pl.ds(NT_FULL * TL * D, TAIL * D)])
        isems = (sem_i0, sem_i1)
        osems = (sem_o0, sem_o1)
        lanes = lax.broadcasted_iota(jnp.int32, (16,), 0)

        def tile_of(i):
            return wid + NW * i

        def start_in(t, buf):
            pltpu.async_copy(tab_hbm.at[:, pl.ds(t * TL, TL)],
                             in_v.at[buf], isems[buf])

        def wait_in(t, buf):
            pltpu.make_async_copy(tab_hbm.at[:, pl.ds(t * TL, TL)],
                                  in_v.at[buf], isems[buf]).wait()

        def transpose_tile(buf):
            for d in range(D):
                vs = [in_v[buf, d, pl.ds(16 * c, 16)]
                      for c in range(TL // 16)]
                for c in range(TL // 16):
                    plsc.store_scatter(
                        pad_v, [lanes + (d * 129 + 16 * c)], vs[c])
            for a in range(D):
                vs = [plsc.load_gather(
                    pad_v,
                    [(lanes + (16 * (g % 2))) * 129 + (4 * a + g // 2)])
                    for g in range(8)]
                for g in range(8):
                    out_v[buf, pl.ds(128 * a + 16 * g, 16)] = vs[g]

        def start_out(t, buf):
            pltpu.async_copy(out_v.at[buf],
                             out_hbm.at[pl.ds(t * TL * D, TL * D)],
                             osems[buf])

        def wait_out(t, buf):
            pltpu.make_async_copy(out_v.at[buf],
                                  out_hbm.at[pl.ds(t * TL * D, TL * D)],
                                  osems[buf]).wait()

        @pl.when(tile_of(0) < NT_FULL)
        def _():
            start_in(tile_of(0), 0)

        def do_tile(i, buf):
            t = tile_of(i)

            @pl.when((i >= 2) & (tile_of(i - 2) < NT_FULL))
            def _():
                wait_out(tile_of(i - 2), buf)

            @pl.when(t < NT_FULL)
            def _():
                tn = tile_of(i + 1)

                @pl.when(tn < NT_FULL)
                def _():
                    start_in(tn, 1 - buf)
                wait_in(t, buf)
                transpose_tile(buf)
                start_out(t, buf)

        n_j = (PER_W + 2) // 2

        def step(j, carry):
            do_tile(2 * j, 0)
            do_tile(2 * j + 1, 1)
            return carry

        lax.fori_loop(0, n_j, step, 0, unroll=False)
        for i in (2 * n_j - 2, 2 * n_j - 1):
            @pl.when(tile_of(i) < NT_FULL)
            def _():
                wait_out(tile_of(i), i % 2)

    return k(table_t, tail_lin)


def _sc_attend(item_input, hist_t, item_table):
    B = item_input.shape[0]
    L = hist_t.shape[0]
    D = item_table.shape[1]
    info = plsc.get_sparse_core_info()
    NC, NS = info.num_cores, info.num_subcores
    NW = NC * NS
    b_per_w = B // NW
    C0, C1 = 128, L - 128
    LP = 16 * ((L + 15) // 16)

    mesh = plsc.VectorSubcoreMesh(core_axis_name="c", subcore_axis_name="s")

    @functools.partial(
        pl.kernel,
        mesh=mesh,
        out_type=(
            jax.ShapeDtypeStruct((B, D), jnp.float32),
            jax.ShapeDtypeStruct((B, D), jnp.float32),
        ),
        scratch_types=[
            pltpu.VMEM((b_per_w,), jnp.int32),
            pltpu.VMEM((b_per_w, D), jnp.float32),
            pltpu.VMEM((L, b_per_w), jnp.int32),
            pltpu.VMEM((b_per_w, LP), jnp.int32),
            pltpu.VMEM((2, L, D), jnp.float32),
            pltpu.VMEM((b_per_w, D), jnp.float32),
            pltpu.SemaphoreType.DMA,
            pltpu.SemaphoreType.DMA,
            pltpu.SemaphoreType.DMA,
        ],
        compiler_params=pltpu.CompilerParams(use_tc_tiling_on_sc=False,
                                             needs_layout_passes=False),
    )
    def k(iidx_hbm, hidx_hbm, itab_hbm, iout, wout,
          iidx_v, irows_v, hcol_v, hidx_v, hrows_v, wout_v,
          sem_i, sem_h0, sem_h1):
        wid = lax.axis_index("s") * NC + lax.axis_index("c")
        base = wid * b_per_w

        pltpu.sync_copy(iidx_hbm.at[pl.ds(base, b_per_w)], iidx_v)
        cp_i = pltpu.async_copy(itab_hbm.at[iidx_v], irows_v, sem_i)
        pltpu.sync_copy(hidx_hbm.at[:, pl.ds(base, b_per_w)], hcol_v)
        lanes = lax.broadcasted_iota(jnp.int32, (16,), 0)

        def transpose_b(b, carry):
            bvec = lax.broadcast_in_dim(b, (16,), ()).astype(jnp.int32)
            for lg in range(LP // 16):
                lvec = lanes + (16 * lg)
                mask = lvec < L
                vals = plsc.load_gather(hcol_v, [lvec, bvec], mask=mask)
                hidx_v[b, pl.ds(16 * lg, 16)] = vals
            return carry

        lax.fori_loop(0, b_per_w, transpose_b, 0, unroll=False)

        sems = (sem_h0, sem_h1)

        def start(b, buf):
            pltpu.async_copy(itab_hbm.at[hidx_v.at[b, pl.ds(0, C0)]],
                             hrows_v.at[buf, pl.ds(0, C0)], sems[buf])
            pltpu.async_copy(itab_hbm.at[hidx_v.at[b, pl.ds(C0, C1)]],
                             hrows_v.at[buf, pl.ds(C0, C1)], sems[buf])

        def drain(b, buf):
            pltpu.make_async_copy(itab_hbm.at[hidx_v.at[b, pl.ds(0, C0)]],
                                  hrows_v.at[buf, pl.ds(0, C0)],
                                  sems[buf]).wait()
            pltpu.make_async_copy(itab_hbm.at[hidx_v.at[b, pl.ds(C0, C1)]],
                                  hrows_v.at[buf, pl.ds(C0, C1)],
                                  sems[buf]).wait()

        start(0, 0)
        cp_i.wait()
        pltpu.sync_copy(irows_v, iout.at[pl.ds(base, b_per_w)])

        def compute_row(b, buf):
            q0 = irows_v[b, pl.ds(0, 16)]
            q1 = irows_v[b, pl.ds(16, 16)]
            zero = jnp.zeros((16,), jnp.float32)

            def group(g, carry):
                a0x, a1x, zx, a0y, a1y, zy = carry
                accs = [[a0x, a1x, zx], [a0y, a1y, zy]]
                for kk in range(8):
                    l = g * 8 + kk
                    h0 = hrows_v[buf, l, pl.ds(0, 16)]
                    h1 = hrows_v[buf, l, pl.ds(16, 16)]
                    s = jnp.sum(h0 * q0 + h1 * q1)
                    e = jnp.exp(lax.broadcast_in_dim(s, (16,), ()))
                    a = accs[kk % 2]
                    a[0] = a[0] + e * h0
                    a[1] = a[1] + e * h1
                    a[2] = a[2] + e
                return (accs[0][0], accs[0][1], accs[0][2],
                        accs[1][0], accs[1][1], accs[1][2])

            a0x, a1x, zx, a0y, a1y, zy = lax.fori_loop(
                0, L // 8, group, (zero, zero, zero, zero, zero, zero),
                unroll=False)
            winv = 1.0 / (zx + zy)
            wout_v[b, pl.ds(0, 16)] = (a0x + a0y) * winv
            wout_v[b, pl.ds(16, 16)] = (a1x + a1y) * winv

        def step(t, carry):
            b = 2 * t
            start(b + 1, 1)
            drain(b, 0)
            compute_row(b, 0)
            start(b + 2, 0)
            drain(b + 1, 1)
            compute_row(b + 1, 1)
            return carry

        lax.fori_loop(0, (b_per_w - 2) // 2, step, 0, unroll=False)
        b = b_per_w - 2
        start(b + 1, 1)
        drain(b, 0)
        compute_row(b, 0)
        drain(b + 1, 1)
        compute_row(b + 1, 1)

        pltpu.sync_copy(wout_v, wout.at[pl.ds(base, b_per_w)])

    return k(item_input, hist_t, item_table)


def _mlp_body(u_ref, i_ref, w_ref, w1_ref, b1_ref, w2_ref, b2_ref,
              w3_ref, b3_ref, wo_ref, bo_ref, out_ref):
    w1 = w1_ref[...]
    x = (jnp.dot(u_ref[...], w1[0:32], preferred_element_type=jnp.float32)
         + jnp.dot(i_ref[...], w1[32:64], preferred_element_type=jnp.float32)
         + jnp.dot(w_ref[...], w1[64:96], preferred_element_type=jnp.float32)
         + b1_ref[...])
    x = jnp.maximum(x, 0.0)
    x = jnp.maximum(jnp.dot(x, w2_ref[...],
                            preferred_element_type=jnp.float32) + b2_ref[...], 0.0)
    x = jnp.maximum(jnp.dot(x, w3_ref[...],
                            preferred_element_type=jnp.float32) + b3_ref[...], 0.0)
    y = jnp.dot(x, wo_ref[...], preferred_element_type=jnp.float32) + bo_ref[...]
    out_ref[...] = jax.nn.sigmoid(y)


def _mlp(user_emb, item_emb, wh, W1, b1, W2, b2, W3, b3, Wo, bo):
    B, D = user_emb.shape
    full = lambda *s: pl.BlockSpec(s, lambda: (0,) * len(s))
    return pl.pallas_call(
        _mlp_body,
        in_specs=[
            full(B, D), full(B, D), full(B, D),
            full(*W1.shape), full(1, b1.shape[0]),
            full(*W2.shape), full(1, b2.shape[0]),
            full(*W3.shape), full(1, b3.shape[0]),
            full(*Wo.shape), full(1, bo.shape[0]),
        ],
        out_specs=full(B, 1),
        out_shape=jax.ShapeDtypeStruct((B, 1), jnp.float32),
    )(user_emb, item_emb, wh, W1, b1.reshape(1, -1), W2,
      b2.reshape(1, -1), W3, b3.reshape(1, -1), Wo, bo.reshape(1, -1))


@jax.jit
def kernel(user_input, item_input, history_items, user_table, item_table,
           W1, b1, W2, b2, W3, b3, Wo, bo):
    hist_t = jnp.transpose(history_items).astype(jnp.int32)
    user_emb = jnp.take(user_table, user_input, axis=0)
    n_tail = item_table.shape[0] % 128
    tail_lin = item_table[item_table.shape[0] - n_tail:].reshape(-1)
    itab_flat = _sc_relayout(jnp.transpose(item_table), tail_lin)
    itab_lin = itab_flat.reshape(item_table.shape)
    item_emb, wh = _sc_attend(item_input.astype(jnp.int32), hist_t,
                              itab_lin)
    return _mlp(user_emb, item_emb, wh, W1, b1, W2, b2, W3, b3, Wo, bo)
